# baseline (speedup 1.0000x reference)
import functools
import math

import jax
import jax.numpy as jnp
from jax import lax
from jax.experimental import pallas as pl
from jax.experimental.pallas import tpu as pltpu
from jax.experimental.pallas import tpu_sc as plsc

D_MODEL = 64
SCALE = math.sqrt(D_MODEL)
LANES = 16
IDXW = 128
GRP = 256
SUB = GRP // IDXW
NG = 4
NS = 2


@functools.cache
def _make(B: int):
    info = plsc.get_sparse_core_info()
    num_workers = info.num_cores * info.num_subcores
    b_per_w = B // num_workers
    n_groups = b_per_w // GRP
    rows_per_w = b_per_w // IDXW
    assert B % num_workers == 0 and b_per_w % GRP == 0 and n_groups % NG == 0
    mesh = plsc.VectorSubcoreMesh(core_axis_name="c", subcore_axis_name="s")

    @functools.partial(
        pl.kernel,
        mesh=mesh,
        out_type=jax.ShapeDtypeStruct((B, D_MODEL), jnp.float32),
        compiler_params=pltpu.CompilerParams(use_tc_tiling_on_sc=False),
        scratch_types=(
            [pltpu.VMEM((rows_per_w, IDXW), jnp.int32)]
            + [pltpu.VMEM((GRP, D_MODEL), jnp.float32) for _ in range(NG + NS)]
            + [pltpu.SemaphoreType.DMA for _ in range(NG + NS)]
        ),
    )
    def k(idx_hbm, table_hbm, out_hbm, idx_v, *bufs_and_sems):
        gbuf = bufs_and_sems[:NG]
        sbuf = bufs_and_sems[NG:NG + NS]
        gsem = bufs_and_sems[NG + NS:2 * NG + NS]
        ssem = bufs_and_sems[2 * NG + NS:]

        wid = lax.axis_index("s") * info.num_cores + lax.axis_index("c")
        row0 = wid * rows_per_w
        base = wid * b_per_w

        pltpu.sync_copy(idx_hbm.at[pl.ds(row0, rows_per_w)], idx_v)

        def fire_gather(g, b):
            for j in range(SUB):
                pltpu.async_copy(
                    table_hbm.at[idx_v.at[g * SUB + j]],
                    gbuf[b].at[pl.ds(j * IDXW, IDXW)],
                    gsem[b],
                )

        def wait_gather(g, b):
            for j in range(SUB):
                pltpu.make_async_copy(
                    table_hbm.at[idx_v.at[g * SUB + j]],
                    gbuf[b].at[pl.ds(j * IDXW, IDXW)],
                    gsem[b],
                ).wait()

        def out_slice(g):
            return out_hbm.at[pl.ds(base + g * GRP, GRP)]

        def fire_store(g, bs):
            pltpu.async_copy(sbuf[bs], out_slice(g), ssem[bs])

        def wait_store(g, bs):
            pltpu.make_async_copy(sbuf[bs], out_slice(g), ssem[bs]).wait()

        def scale(b, bs):
            def row_body(r, c):
                for j in range(D_MODEL // LANES):
                    sl = pl.ds(j * LANES, LANES)
                    sbuf[bs][r, sl] = gbuf[b][r, sl] * SCALE
                return c

            lax.fori_loop(0, GRP, row_body, 0, unroll=4)

        for b in range(NG):
            fire_gather(b, b)

        def outer(i0, carry):
            for b in range(NG):
                g = i0 * NG + b
                bs = b % NS
                wait_gather(g, b)

                @pl.when(jnp.logical_or(i0 > 0, b >= NS))
                def _():
                    wait_store(g - NS, bs)

                scale(b, bs)
                fire_store(g, bs)

                @pl.when(i0 < n_groups // NG - 1)
                def _():
                    fire_gather(g + NG, b)

            return carry

        lax.fori_loop(0, n_groups // NG, outer, 0)
        for b in range(NS):
            wait_store(n_groups - NS + b, (n_groups - NS + b) % NS)

    return k


def kernel(x, table):
    B = x.shape[0] * x.shape[1]
    out = _make(B)(x.reshape(B // IDXW, IDXW), table)
    return out.reshape(x.shape[0], x.shape[1], D_MODEL)
